# baseline (speedup 1.0000x reference)
import functools
import math

import jax
import jax.numpy as jnp
from jax.experimental import pallas as pl
from jax.experimental.pallas import tpu as pltpu

HIDDEN = 128
VMEM_LIMIT = 56 * 1024 * 1024
NB = 8


def _round_up(x, m):
    return (x + m - 1) // m * m


NB0 = 8


def _conv0_kernel(p_ref, w_ref, b_ref, o_ref, *, nb):
    m = nb * 1024
    lhs = p_ref[...].reshape(m, p_ref.shape[-1])
    acc = (jnp.dot(lhs, w_ref[...], preferred_element_type=jnp.float32)
           + b_ref[...])
    o_ref[...] = jnp.maximum(acc, 0.0).astype(jnp.bfloat16).reshape(
        nb, 1024, o_ref.shape[-1])


def _conv0(xs, w, bias):
    N = xs.shape[0]
    xp = jnp.pad(xs, ((0, 0), (3, 3), (3, 3), (0, 0)))
    taps = [xp[:, dy:dy + 64:2, dx:dx + 64:2, :] for dy in range(7)
            for dx in range(7)]
    patches = jnp.concatenate(taps, axis=-1).reshape(N, 1024, 294)
    w2d = w.reshape(294, 64)

    return pl.pallas_call(
        functools.partial(_conv0_kernel, nb=NB0),
        out_shape=jax.ShapeDtypeStruct((N, 1024, 64), jnp.bfloat16),
        grid_spec=pltpu.PrefetchScalarGridSpec(
            num_scalar_prefetch=0,
            grid=(N // NB0,),
            in_specs=[
                pl.BlockSpec((NB0, 1024, 294), lambda i: (i, 0, 0)),
                pl.BlockSpec((294, 64), lambda i: (0, 0)),
                pl.BlockSpec((1, 64), lambda i: (0, 0)),
            ],
            out_specs=pl.BlockSpec((NB0, 1024, 64), lambda i: (i, 0, 0)),
        ),
        compiler_params=pltpu.CompilerParams(
            dimension_semantics=("parallel",),
            vmem_limit_bytes=VMEM_LIMIT),
    )(patches, w2d, bias)


def _conv_s1(h, w_ref, b_ref, nb, hw, cin, cout):
    hp = jnp.pad(h, ((0, 0), (1, 1), (1, 1), (0, 0)))
    m = nb * hw * hw
    acc = jnp.broadcast_to(b_ref[...], (m, cout))
    for dy in range(3):
        for dx in range(3):
            lhs = hp[:, dy:dy + hw, dx:dx + hw, :].reshape(m, cin)
            acc = acc + jnp.dot(lhs, w_ref[dy * 3 + dx],
                                preferred_element_type=jnp.float32)
    return jnp.maximum(acc, 0.0).astype(jnp.bfloat16).reshape(nb, hw, hw, cout)


def _conv_s2(h, w_ref, b_ref, nb, hw, cin, cout, k=3):
    p = (k - 1) // 2
    ho = hw // 2
    hp = jnp.pad(h, ((0, 0), (p, p), (p, p), (0, 0)))
    pdim = hw + 2 * p
    r = pdim // 2
    he = hp.reshape(nb, r, 2, pdim, cin)
    rows = (he[:, :, 0], he[:, :, 1])
    phases = tuple(
        tuple(row.reshape(nb, r, r, 2, cin)[:, :, :, q] for q in (0, 1))
        for row in rows)
    m = nb * ho * ho
    acc = jnp.broadcast_to(b_ref[...], (m, cout))
    for dy in range(k):
        for dx in range(k):
            g = phases[dy & 1][dx & 1]
            qy, qx = dy >> 1, dx >> 1
            lhs = g[:, qy:qy + ho, qx:qx + ho, :].reshape(m, cin)
            acc = acc + jnp.dot(lhs, w_ref[dy * k + dx],
                                preferred_element_type=jnp.float32)
    return jnp.maximum(acc, 0.0).astype(jnp.bfloat16).reshape(nb, ho, ho, cout)


def _tail_kernel(x_ref, w2, b2c, w3, b3c, w4, b4, w5, b5, w6, b6, w7, b7,
                 w8, b8, w9, b9, wih1_ref, o_ref, *, nb):
    h = x_ref[...].reshape(nb, 32, 32, 64)
    h = _conv_s2(h, w2, b2c, nb, 32, 64, 128, k=5)
    h = _conv_s2(h, w3, b3c, nb, 16, 128, 256, k=5)
    h = _conv_s1(h, w4, b4, nb, 8, 256, 256)
    h = _conv_s2(h, w5, b5, nb, 8, 256, 512)
    h = _conv_s1(h, w6, b6, nb, 4, 512, 512)
    h = _conv_s2(h, w7, b7, nb, 4, 512, 512)
    h = _conv_s1(h, w8, b8, nb, 2, 512, 512)
    h = _conv_s2(h, w9, b9, nb, 2, 512, 1024)
    feat = h.reshape(nb, 1024)
    o_ref[...] = jnp.dot(feat, wih1_ref[...],
                         preferred_element_type=jnp.float32)


def _tail(h1, tail_ws, tail_bs, wih1):
    N = h1.shape[0]
    x = h1.reshape(N, 1024, 64)
    full = lambda shape: pl.BlockSpec(shape, lambda i: (0,) * len(shape))
    in_specs = [pl.BlockSpec((NB, 1024, 64), lambda i: (i, 0, 0))]
    args = [x]
    for w, b in zip(tail_ws, tail_bs):
        in_specs += [full(w.shape), full(b.shape)]
        args += [w, b]
    in_specs.append(full(wih1.shape))
    args.append(wih1)

    return pl.pallas_call(
        functools.partial(_tail_kernel, nb=NB),
        out_shape=jax.ShapeDtypeStruct((N, 4 * HIDDEN), jnp.float32),
        grid_spec=pltpu.PrefetchScalarGridSpec(
            num_scalar_prefetch=0,
            grid=(N // NB,),
            in_specs=in_specs,
            out_specs=pl.BlockSpec((NB, 4 * HIDDEN), lambda i: (i, 0)),
        ),
        compiler_params=pltpu.CompilerParams(
            dimension_semantics=("parallel",),
            vmem_limit_bytes=VMEM_LIMIT),
    )(*args)


def _lstm_kernel(xg_ref, whh1_ref, b1_ref, wih2_ref, whh2_ref, b2_ref,
                 wlin_ref, blin_ref, out_ref, h1_sc, c1_sc, h2_sc, c2_sc,
                 *, hidden):
    t = pl.program_id(0)

    @pl.when(t == 0)
    def _():
        h1_sc[...] = jnp.zeros_like(h1_sc)
        c1_sc[...] = jnp.zeros_like(c1_sc)
        h2_sc[...] = jnp.zeros_like(h2_sc)
        c2_sc[...] = jnp.zeros_like(c2_sc)

    H = hidden

    def cell(gx, whh_ref, b_ref, h_sc, c_sc):
        gates = (gx
                 + jnp.dot(h_sc[...].astype(jnp.bfloat16), whh_ref[...],
                           preferred_element_type=jnp.float32)
                 + b_ref[...])
        i = jax.nn.sigmoid(gates[:, 0 * H:1 * H])
        f = jax.nn.sigmoid(gates[:, 1 * H:2 * H])
        g = jnp.tanh(gates[:, 2 * H:3 * H])
        o = jax.nn.sigmoid(gates[:, 3 * H:4 * H])
        c = f * c_sc[...] + i * g
        h = o * jnp.tanh(c)
        c_sc[...] = c
        h_sc[...] = h
        return h

    h1 = cell(xg_ref[0], whh1_ref, b1_ref, h1_sc, c1_sc)
    gx2 = jnp.dot(h1.astype(jnp.bfloat16), wih2_ref[...],
                  preferred_element_type=jnp.float32)
    h2 = cell(gx2, whh2_ref, b2_ref, h2_sc, c2_sc)
    out_ref[0] = (jnp.dot(h2, wlin_ref[...], preferred_element_type=jnp.float32)
                  + blin_ref[...])


def _lstm_head(xg1, B, seq, whh1, b1, wih2, whh2, b2, wlin, blin):
    H = HIDDEN
    Bp = _round_up(max(B, 8), 8)
    xg = xg1.reshape(B, seq, 4 * H).transpose(1, 0, 2)
    xg = jnp.pad(xg, ((0, 0), (0, Bp - B), (0, 0)))

    out = pl.pallas_call(
        functools.partial(_lstm_kernel, hidden=H),
        out_shape=jax.ShapeDtypeStruct((seq, Bp, 6), jnp.float32),
        grid_spec=pltpu.PrefetchScalarGridSpec(
            num_scalar_prefetch=0,
            grid=(seq,),
            in_specs=[
                pl.BlockSpec((1, Bp, 4 * H), lambda t: (t, 0, 0)),
                pl.BlockSpec((H, 4 * H), lambda t: (0, 0)),
                pl.BlockSpec((1, 4 * H), lambda t: (0, 0)),
                pl.BlockSpec((H, 4 * H), lambda t: (0, 0)),
                pl.BlockSpec((H, 4 * H), lambda t: (0, 0)),
                pl.BlockSpec((1, 4 * H), lambda t: (0, 0)),
                pl.BlockSpec((H, 6), lambda t: (0, 0)),
                pl.BlockSpec((1, 6), lambda t: (0, 0)),
            ],
            out_specs=pl.BlockSpec((1, Bp, 6), lambda t: (t, 0, 0)),
            scratch_shapes=[pltpu.VMEM((Bp, H), jnp.float32)] * 4,
        ),
        compiler_params=pltpu.CompilerParams(
            dimension_semantics=("arbitrary",),
            vmem_limit_bytes=VMEM_LIMIT),
    )(xg, whh1, b1, wih2, whh2, b2, wlin, blin)
    return out[:, :B, :].transpose(1, 0, 2)


def kernel(x,
           conv0_w, conv0_b, conv1_w, conv1_b, conv2_w, conv2_b,
           conv3_w, conv3_b, conv4_w, conv4_b, conv5_w, conv5_b,
           conv6_w, conv6_b, conv7_w, conv7_b, conv8_w, conv8_b,
           wih1, whh1, b1, wih2, whh2, b2, wlin, blin):
    B, T, _, H, W = x.shape
    seq = T - 1
    xs = jnp.concatenate([x[:, :-1], x[:, 1:]], axis=2)
    xs = xs.reshape(B * seq, 6, H, W).transpose(0, 2, 3, 1).astype(jnp.bfloat16)

    h = _conv0(xs, conv0_w, conv0_b)

    xg1 = _tail(h,
                (conv1_w, conv2_w, conv3_w, conv4_w, conv5_w,
                 conv6_w, conv7_w, conv8_w),
                (conv1_b, conv2_b, conv3_b, conv4_b, conv5_b,
                 conv6_b, conv7_b, conv8_b),
                wih1)

    return _lstm_head(xg1, B, seq, whh1, b1, wih2, whh2, b2, wlin, blin)


# speedup vs baseline: 144.3229x; 144.3229x over previous
import functools
import math

import jax
import jax.numpy as jnp
from jax.experimental import pallas as pl
from jax.experimental.pallas import tpu as pltpu

HIDDEN = 128
VMEM_LIMIT = 56 * 1024 * 1024
NB = 8


def _round_up(x, m):
    return (x + m - 1) // m * m


NB0 = 8


def _conv0_kernel(p_ref, w_ref, b_ref, o_ref, *, nb):
    m = nb * 1024
    lhs = p_ref[...].reshape(m, p_ref.shape[-1])
    acc = (jnp.dot(lhs, w_ref[...], preferred_element_type=jnp.float32)
           + b_ref[...])
    o_ref[...] = jnp.maximum(acc, 0.0).astype(jnp.bfloat16).reshape(
        nb, 1024, o_ref.shape[-1])


def _conv0(xs, w, bias):
    N = xs.shape[0]
    xp = jnp.pad(xs, ((0, 0), (3, 3), (3, 3), (0, 0)))
    ph = xp.reshape(N, 35, 2, 35, 2, 6).transpose(0, 1, 3, 2, 4, 5)
    ph = ph.reshape(N, 35, 35, 24)
    taps = [ph[:, qy:qy + 32, qx:qx + 32, :] for qy in range(4)
            for qx in range(4)]
    patches = jnp.concatenate(taps, axis=-1).reshape(N, 1024, 384)
    w8 = jnp.zeros((8, 8, 6, 64), w.dtype).at[:7, :7].set(w.reshape(7, 7, 6, 64))
    w2d = w8.reshape(4, 2, 4, 2, 6, 64).transpose(0, 2, 1, 3, 4, 5).reshape(384, 64)

    return pl.pallas_call(
        functools.partial(_conv0_kernel, nb=NB0),
        out_shape=jax.ShapeDtypeStruct((N, 1024, 64), jnp.bfloat16),
        grid_spec=pltpu.PrefetchScalarGridSpec(
            num_scalar_prefetch=0,
            grid=(N // NB0,),
            in_specs=[
                pl.BlockSpec((NB0, 1024, 384), lambda i: (i, 0, 0)),
                pl.BlockSpec((384, 64), lambda i: (0, 0)),
                pl.BlockSpec((1, 64), lambda i: (0, 0)),
            ],
            out_specs=pl.BlockSpec((NB0, 1024, 64), lambda i: (i, 0, 0)),
        ),
        compiler_params=pltpu.CompilerParams(
            dimension_semantics=("parallel",),
            vmem_limit_bytes=VMEM_LIMIT),
    )(patches, w2d, bias)


def _conv_s1(h, w_ref, b_ref, nb, hw, cin, cout):
    hp = jnp.pad(h, ((0, 0), (1, 1), (1, 1), (0, 0)))
    m = nb * hw * hw
    acc = jnp.broadcast_to(b_ref[...], (m, cout))
    for dy in range(3):
        for dx in range(3):
            lhs = hp[:, dy:dy + hw, dx:dx + hw, :].reshape(m, cin)
            acc = acc + jnp.dot(lhs, w_ref[dy * 3 + dx],
                                preferred_element_type=jnp.float32)
    return jnp.maximum(acc, 0.0).astype(jnp.bfloat16).reshape(nb, hw, hw, cout)


def _conv_s2(h, w_ref, b_ref, nb, hw, cin, cout, k=3):
    p = (k - 1) // 2
    ho = hw // 2
    hp = jnp.pad(h, ((0, 0), (p, p), (p, p), (0, 0)))
    pdim = hw + 2 * p
    r = pdim // 2
    he = hp.reshape(nb, r, 2, pdim, cin)
    rows = (he[:, :, 0], he[:, :, 1])
    phases = tuple(
        tuple(row.reshape(nb, r, r, 2, cin)[:, :, :, q] for q in (0, 1))
        for row in rows)
    m = nb * ho * ho
    acc = jnp.broadcast_to(b_ref[...], (m, cout))
    for dy in range(k):
        for dx in range(k):
            g = phases[dy & 1][dx & 1]
            qy, qx = dy >> 1, dx >> 1
            lhs = g[:, qy:qy + ho, qx:qx + ho, :].reshape(m, cin)
            acc = acc + jnp.dot(lhs, w_ref[dy * k + dx],
                                preferred_element_type=jnp.float32)
    return jnp.maximum(acc, 0.0).astype(jnp.bfloat16).reshape(nb, ho, ho, cout)


def _tail_kernel(x_ref, w2, b2c, w3, b3c, w4, b4, w5, b5, w6, b6, w7, b7,
                 w8, b8, w9, b9, wih1_ref, o_ref, *, nb):
    h = x_ref[...].reshape(nb, 32, 32, 64)
    h = _conv_s2(h, w2, b2c, nb, 32, 64, 128, k=5)
    h = _conv_s2(h, w3, b3c, nb, 16, 128, 256, k=5)
    h = _conv_s1(h, w4, b4, nb, 8, 256, 256)
    h = _conv_s2(h, w5, b5, nb, 8, 256, 512)
    h = _conv_s1(h, w6, b6, nb, 4, 512, 512)
    h = _conv_s2(h, w7, b7, nb, 4, 512, 512)
    h = _conv_s1(h, w8, b8, nb, 2, 512, 512)
    h = _conv_s2(h, w9, b9, nb, 2, 512, 1024)
    feat = h.reshape(nb, 1024)
    o_ref[...] = jnp.dot(feat, wih1_ref[...],
                         preferred_element_type=jnp.float32)


def _tail(h1, tail_ws, tail_bs, wih1):
    N = h1.shape[0]
    x = h1.reshape(N, 1024, 64)
    full = lambda shape: pl.BlockSpec(shape, lambda i: (0,) * len(shape))
    in_specs = [pl.BlockSpec((NB, 1024, 64), lambda i: (i, 0, 0))]
    args = [x]
    for w, b in zip(tail_ws, tail_bs):
        in_specs += [full(w.shape), full(b.shape)]
        args += [w, b]
    in_specs.append(full(wih1.shape))
    args.append(wih1)

    return pl.pallas_call(
        functools.partial(_tail_kernel, nb=NB),
        out_shape=jax.ShapeDtypeStruct((N, 4 * HIDDEN), jnp.float32),
        grid_spec=pltpu.PrefetchScalarGridSpec(
            num_scalar_prefetch=0,
            grid=(N // NB,),
            in_specs=in_specs,
            out_specs=pl.BlockSpec((NB, 4 * HIDDEN), lambda i: (i, 0)),
        ),
        compiler_params=pltpu.CompilerParams(
            dimension_semantics=("parallel",),
            vmem_limit_bytes=VMEM_LIMIT),
    )(*args)


def _lstm_kernel(xg_ref, whh1_ref, b1_ref, wih2_ref, whh2_ref, b2_ref,
                 wlin_ref, blin_ref, out_ref, h1_sc, c1_sc, h2_sc, c2_sc,
                 *, hidden):
    t = pl.program_id(0)

    @pl.when(t == 0)
    def _():
        h1_sc[...] = jnp.zeros_like(h1_sc)
        c1_sc[...] = jnp.zeros_like(c1_sc)
        h2_sc[...] = jnp.zeros_like(h2_sc)
        c2_sc[...] = jnp.zeros_like(c2_sc)

    H = hidden

    def cell(gx, whh_ref, b_ref, h_sc, c_sc):
        gates = (gx
                 + jnp.dot(h_sc[...].astype(jnp.bfloat16), whh_ref[...],
                           preferred_element_type=jnp.float32)
                 + b_ref[...])
        i = jax.nn.sigmoid(gates[:, 0 * H:1 * H])
        f = jax.nn.sigmoid(gates[:, 1 * H:2 * H])
        g = jnp.tanh(gates[:, 2 * H:3 * H])
        o = jax.nn.sigmoid(gates[:, 3 * H:4 * H])
        c = f * c_sc[...] + i * g
        h = o * jnp.tanh(c)
        c_sc[...] = c
        h_sc[...] = h
        return h

    h1 = cell(xg_ref[0], whh1_ref, b1_ref, h1_sc, c1_sc)
    gx2 = jnp.dot(h1.astype(jnp.bfloat16), wih2_ref[...],
                  preferred_element_type=jnp.float32)
    h2 = cell(gx2, whh2_ref, b2_ref, h2_sc, c2_sc)
    out_ref[0] = (jnp.dot(h2, wlin_ref[...], preferred_element_type=jnp.float32)
                  + blin_ref[...])


def _lstm_head(xg1, B, seq, whh1, b1, wih2, whh2, b2, wlin, blin):
    H = HIDDEN
    Bp = _round_up(max(B, 8), 8)
    xg = xg1.reshape(B, seq, 4 * H).transpose(1, 0, 2)
    xg = jnp.pad(xg, ((0, 0), (0, Bp - B), (0, 0)))

    out = pl.pallas_call(
        functools.partial(_lstm_kernel, hidden=H),
        out_shape=jax.ShapeDtypeStruct((seq, Bp, 6), jnp.float32),
        grid_spec=pltpu.PrefetchScalarGridSpec(
            num_scalar_prefetch=0,
            grid=(seq,),
            in_specs=[
                pl.BlockSpec((1, Bp, 4 * H), lambda t: (t, 0, 0)),
                pl.BlockSpec((H, 4 * H), lambda t: (0, 0)),
                pl.BlockSpec((1, 4 * H), lambda t: (0, 0)),
                pl.BlockSpec((H, 4 * H), lambda t: (0, 0)),
                pl.BlockSpec((H, 4 * H), lambda t: (0, 0)),
                pl.BlockSpec((1, 4 * H), lambda t: (0, 0)),
                pl.BlockSpec((H, 6), lambda t: (0, 0)),
                pl.BlockSpec((1, 6), lambda t: (0, 0)),
            ],
            out_specs=pl.BlockSpec((1, Bp, 6), lambda t: (t, 0, 0)),
            scratch_shapes=[pltpu.VMEM((Bp, H), jnp.float32)] * 4,
        ),
        compiler_params=pltpu.CompilerParams(
            dimension_semantics=("arbitrary",),
            vmem_limit_bytes=VMEM_LIMIT),
    )(xg, whh1, b1, wih2, whh2, b2, wlin, blin)
    return out[:, :B, :].transpose(1, 0, 2)


def kernel(x,
           conv0_w, conv0_b, conv1_w, conv1_b, conv2_w, conv2_b,
           conv3_w, conv3_b, conv4_w, conv4_b, conv5_w, conv5_b,
           conv6_w, conv6_b, conv7_w, conv7_b, conv8_w, conv8_b,
           wih1, whh1, b1, wih2, whh2, b2, wlin, blin):
    B, T, _, H, W = x.shape
    seq = T - 1
    xs = jnp.concatenate([x[:, :-1], x[:, 1:]], axis=2)
    xs = xs.reshape(B * seq, 6, H, W).transpose(0, 2, 3, 1).astype(jnp.bfloat16)

    h = _conv0(xs, conv0_w, conv0_b)

    xg1 = _tail(h,
                (conv1_w, conv2_w, conv3_w, conv4_w, conv5_w,
                 conv6_w, conv7_w, conv8_w),
                (conv1_b, conv2_b, conv3_b, conv4_b, conv5_b,
                 conv6_b, conv7_b, conv8_b),
                wih1)

    return _lstm_head(xg1, B, seq, whh1, b1, wih2, whh2, b2, wlin, blin)
